# Initial kernel scaffold; baseline (speedup 1.0000x reference)
import jax
import jax.numpy as jnp
from jax.experimental import pallas as pl


def kernel(x, edge_index, batch, W1, b1, W2, b2):
    raise NotImplementedError("write your pallas kernel here")


# speedup vs baseline: 21.3365x; 21.3365x over previous
import functools

import jax
import jax.numpy as jnp
from jax import lax
from jax.experimental import pallas as pl
from jax.experimental.pallas import tpu as pltpu
from jax.experimental.pallas import tpu_sc as plsc

N_NODES = 10000
N_EDGES = 320000
D = 128
DH = D // 2
N_GRAPHS = 64

NC = 2
NS = 16
E_PER_TILE = N_EDGES // NS
CHUNK = 125
NCHUNK_AGG = E_PER_TILE // CHUNK
E_PER_W = N_EDGES // (NC * NS)
NCHUNK_DEG = E_PER_W // CHUNK
WB = 624
ZCHUNK = 208
TAIL_OFF = NS * WB
TAIL = N_NODES - TAIL_OFF

_MESH = plsc.VectorSubcoreMesh(
    core_axis_name="c", subcore_axis_name="s", num_cores=NC, num_subcores=NS)
_SC_PARAMS = pltpu.CompilerParams(use_tc_tiling_on_sc=False)


def _fill(ref, nrow, ncol, val):
    v = jnp.full((16,), val, jnp.float32)

    def body(i, _):
        for k in range(ncol // 16):
            ref[i, pl.ds(k * 16, 16)] = v
        return 0

    lax.fori_loop(0, nrow, body, 0)


def _zero_acc(zbuf, acc_sh, sid, ncol):
    _fill(zbuf, ZCHUNK, ncol, 0.0)
    for t in range(3):
        pltpu.sync_copy(zbuf, acc_sh.at[pl.ds(sid * WB + t * ZCHUNK, ZCHUNK)])

    @pl.when(sid == 0)
    def _():
        pltpu.sync_copy(zbuf.at[pl.ds(0, TAIL)], acc_sh.at[pl.ds(TAIL_OFF, TAIL)])


def _writeback(acc_sh, out_hbm, cid, sid):
    rows = pl.ds(sid * WB, WB)
    pltpu.sync_copy(acc_sh.at[rows], out_hbm.at[cid, rows])

    @pl.when(sid == 0)
    def _():
        tl = pl.ds(TAIL_OFF, TAIL)
        pltpu.sync_copy(acc_sh.at[tl], out_hbm.at[cid, tl])


def _sc_deg_body(dst_hbm, out_hbm, idx_v, ones_v, zbuf, acc_sh, sem):
    cid = lax.axis_index("c")
    sid = lax.axis_index("s")
    _zero_acc(zbuf, acc_sh, sid, 16)
    plsc.subcore_barrier()

    _fill(ones_v, CHUNK, 16, 1.0)
    pltpu.sync_copy(dst_hbm.at[cid, sid], idx_v)

    def body(j, _):
        pltpu.sync_copy(ones_v, acc_sh.at[idx_v.at[j]], add=True)
        return 0

    lax.fori_loop(0, NCHUNK_DEG, body, 0)
    plsc.subcore_barrier()
    _writeback(acc_sh, out_hbm, cid, sid)


@functools.partial(
    pl.kernel,
    out_type=jax.ShapeDtypeStruct((NC, N_NODES, 16), jnp.float32),
    mesh=_MESH,
    scratch_types=[
        pltpu.VMEM((NCHUNK_DEG, CHUNK), jnp.int32),
        pltpu.VMEM((CHUNK, 16), jnp.float32),
        pltpu.VMEM((ZCHUNK, 16), jnp.float32),
        pltpu.VMEM_SHARED((N_NODES, 16), jnp.float32),
        pltpu.SemaphoreType.DMA,
    ],
    compiler_params=_SC_PARAMS,
)
def _sc_deg(dst_hbm, out_hbm, idx_v, ones_v, zbuf, acc_sh, sem):
    _sc_deg_body(dst_hbm, out_hbm, idx_v, ones_v, zbuf, acc_sh, sem)


def _sc_agg_body(y_hbm, src_hbm, dst_hbm, out_hbm,
                 idx_s, idx_d, buf0, buf1, zbuf, acc_sh, sem0, sem1):
    cid = lax.axis_index("c")
    sid = lax.axis_index("s")
    _zero_acc(zbuf, acc_sh, sid, DH)
    plsc.subcore_barrier()

    pltpu.sync_copy(src_hbm.at[sid], idx_s)
    pltpu.sync_copy(dst_hbm.at[sid], idx_d)

    pltpu.async_copy(y_hbm.at[cid].at[idx_s.at[0]], buf0, sem0)

    def body(j, _):
        @pl.when(j % 2 == 0)
        def _():
            @pl.when(j + 1 < NCHUNK_AGG)
            def _():
                pltpu.async_copy(y_hbm.at[cid].at[idx_s.at[j + 1]], buf1, sem1)
            pltpu.make_async_copy(y_hbm.at[cid].at[idx_s.at[0]], buf0, sem0).wait()
            pltpu.sync_copy(buf0, acc_sh.at[idx_d.at[j]], add=True)

        @pl.when(j % 2 == 1)
        def _():
            @pl.when(j + 1 < NCHUNK_AGG)
            def _():
                pltpu.async_copy(y_hbm.at[cid].at[idx_s.at[j + 1]], buf0, sem0)
            pltpu.make_async_copy(y_hbm.at[cid].at[idx_s.at[0]], buf1, sem1).wait()
            pltpu.sync_copy(buf1, acc_sh.at[idx_d.at[j]], add=True)

        return 0

    lax.fori_loop(0, NCHUNK_AGG, body, 0)
    plsc.subcore_barrier()
    _writeback(acc_sh, out_hbm, cid, sid)


@functools.partial(
    pl.kernel,
    out_type=jax.ShapeDtypeStruct((NC, N_NODES, DH), jnp.float32),
    mesh=_MESH,
    scratch_types=[
        pltpu.VMEM((NCHUNK_AGG, CHUNK), jnp.int32),
        pltpu.VMEM((NCHUNK_AGG, CHUNK), jnp.int32),
        pltpu.VMEM((CHUNK, DH), jnp.float32),
        pltpu.VMEM((CHUNK, DH), jnp.float32),
        pltpu.VMEM((ZCHUNK, DH), jnp.float32),
        pltpu.VMEM_SHARED((N_NODES, DH), jnp.float32),
        pltpu.SemaphoreType.DMA,
        pltpu.SemaphoreType.DMA,
    ],
    compiler_params=_SC_PARAMS,
)
def _sc_agg(y_hbm, src_hbm, dst_hbm, out_hbm,
            idx_s, idx_d, buf0, buf1, zbuf, acc_sh, sem0, sem1):
    _sc_agg_body(y_hbm, src_hbm, dst_hbm, out_hbm,
                 idx_s, idx_d, buf0, buf1, zbuf, acc_sh, sem0, sem1)


ROW_BLK = 400
N_BLK = N_NODES // ROW_BLK


def _dinv_from_degs(degs_blk):
    deg = degs_blk[0, :, 0] + degs_blk[1, :, 0] + 1.0
    return lax.rsqrt(deg)


def _split_store(ref, val):
    ref[0] = val[:, :DH]
    ref[1] = val[:, DH:]


def _cat(ref):
    return jnp.concatenate([ref[0], ref[1]], axis=1)


def _tc_first_body(x_ref, w_ref, degs_ref, y_ref):
    xw = jnp.dot(x_ref[...], w_ref[...], preferred_element_type=jnp.float32)
    dinv = _dinv_from_degs(degs_ref[...])
    _split_store(y_ref, xw * dinv[:, None])


def _tc_first(x, W1, degs):
    return pl.pallas_call(
        _tc_first_body,
        grid=(N_BLK,),
        in_specs=[
            pl.BlockSpec((ROW_BLK, D), lambda i: (i, 0)),
            pl.BlockSpec((D, D), lambda i: (0, 0)),
            pl.BlockSpec((NC, ROW_BLK, 16), lambda i: (0, i, 0)),
        ],
        out_specs=pl.BlockSpec((NC, ROW_BLK, DH), lambda i: (0, i, 0)),
        out_shape=jax.ShapeDtypeStruct((NC, N_NODES, DH), jnp.float32),
    )(x, W1, degs)


def _tc_mid_body(acc_ref, y1_ref, degs_ref, b1_ref, w2_ref, y2_ref):
    dinv = _dinv_from_degs(degs_ref[...])
    h = (_cat(acc_ref) + _cat(y1_ref)) * dinv[:, None] + b1_ref[...]
    h = jnp.maximum(h, 0.0)
    hw2 = jnp.dot(h, w2_ref[...], preferred_element_type=jnp.float32)
    _split_store(y2_ref, hw2 * dinv[:, None])


def _tc_mid(acc1, y1, degs, b1, W2):
    return pl.pallas_call(
        _tc_mid_body,
        grid=(N_BLK,),
        in_specs=[
            pl.BlockSpec((NC, ROW_BLK, DH), lambda i: (0, i, 0)),
            pl.BlockSpec((NC, ROW_BLK, DH), lambda i: (0, i, 0)),
            pl.BlockSpec((NC, ROW_BLK, 16), lambda i: (0, i, 0)),
            pl.BlockSpec((1, D), lambda i: (0, 0)),
            pl.BlockSpec((D, D), lambda i: (0, 0)),
        ],
        out_specs=pl.BlockSpec((NC, ROW_BLK, DH), lambda i: (0, i, 0)),
        out_shape=jax.ShapeDtypeStruct((NC, N_NODES, DH), jnp.float32),
    )(acc1, y1, degs, b1, W2)


def _tc_pool_body(acc_ref, y2_ref, degs_ref, b2_ref, batch_ref, out_ref,
                  sums, cnts):
    i = pl.program_id(0)

    @pl.when(i == 0)
    def _():
        sums[...] = jnp.zeros((N_GRAPHS, D), jnp.float32)
        cnts[...] = jnp.zeros((N_GRAPHS, D), jnp.float32)

    dinv = _dinv_from_degs(degs_ref[...])
    node = (_cat(acc_ref) + _cat(y2_ref)) * dinv[:, None] + b2_ref[...]
    bb = batch_ref[0, 0, :]
    gids = lax.broadcasted_iota(jnp.int32, (ROW_BLK, N_GRAPHS), 1)
    oh = (bb[:, None] == gids).astype(jnp.float32)
    dn = (((0,), (0,)), ((), ()))
    sums[...] += lax.dot_general(oh, node, dn, preferred_element_type=jnp.float32)
    cnts[...] += lax.dot_general(oh, jnp.ones((ROW_BLK, D), jnp.float32), dn,
                                 preferred_element_type=jnp.float32)

    @pl.when(i == N_BLK - 1)
    def _():
        out_ref[...] = sums[...] / jnp.maximum(cnts[...], 1.0)


def _tc_pool(acc2, y2, degs, b2, batch_r):
    return pl.pallas_call(
        _tc_pool_body,
        grid=(N_BLK,),
        in_specs=[
            pl.BlockSpec((NC, ROW_BLK, DH), lambda i: (0, i, 0)),
            pl.BlockSpec((NC, ROW_BLK, DH), lambda i: (0, i, 0)),
            pl.BlockSpec((NC, ROW_BLK, 16), lambda i: (0, i, 0)),
            pl.BlockSpec((1, D), lambda i: (0, 0)),
            pl.BlockSpec((1, 1, ROW_BLK), lambda i: (i, 0, 0)),
        ],
        out_specs=pl.BlockSpec((N_GRAPHS, D), lambda i: (0, 0)),
        out_shape=jax.ShapeDtypeStruct((N_GRAPHS, D), jnp.float32),
        scratch_shapes=[
            pltpu.VMEM((N_GRAPHS, D), jnp.float32),
            pltpu.VMEM((N_GRAPHS, D), jnp.float32),
        ],
    )(acc2, y2, degs, b2, batch_r)


@jax.jit
def kernel(x, edge_index, batch, W1, b1, W2, b2):
    src32 = edge_index[0].astype(jnp.int32)
    dst32 = edge_index[1].astype(jnp.int32)
    src_a = src32.reshape(NS, NCHUNK_AGG, CHUNK)
    dst_a = dst32.reshape(NS, NCHUNK_AGG, CHUNK)
    dst_d = dst32.reshape(NC, NS, NCHUNK_DEG, CHUNK)
    batch_r = batch.astype(jnp.int32).reshape(N_BLK, 1, ROW_BLK)
    b1r = b1.reshape(1, D)
    b2r = b2.reshape(1, D)

    degs = _sc_deg(dst_d)
    y1 = _tc_first(x, W1, degs)
    acc1 = _sc_agg(y1, src_a, dst_a)
    y2 = _tc_mid(acc1, y1, degs, b1r, W2)
    acc2 = _sc_agg(y2, src_a, dst_a)
    return _tc_pool(acc2, y2, degs, b2r, batch_r)
